# Initial kernel scaffold; baseline (speedup 1.0000x reference)
import jax
import jax.numpy as jnp
from jax.experimental import pallas as pl


def kernel(x, edge_index, W, b, gamma, beta):
    raise NotImplementedError("write your pallas kernel here")


# speedup vs baseline: 3.4427x; 3.4427x over previous
import jax
import jax.numpy as jnp
from jax import lax
from jax.experimental import pallas as pl
from jax.experimental.pallas import tpu as pltpu
from jax.experimental.pallas import tpu_sc as plsc

_NCORES = 2
_NSUB = 16
_NW = _NCORES * _NSUB
_CHUNK = 128


def _mesh():
    return plsc.VectorSubcoreMesh(
        core_axis_name="c", subcore_axis_name="s",
        num_cores=_NCORES, num_subcores=_NSUB)


def _stripe_init(zbuf, shared, base, full_chunks, tail):
    @pl.loop(0, full_chunks)
    def _(k):
        pltpu.sync_copy(zbuf, shared.at[pl.ds(base + k * _CHUNK, _CHUNK)])
    if tail:
        pltpu.sync_copy(zbuf.at[pl.ds(0, tail)],
                        shared.at[pl.ds(base + full_chunks * _CHUNK, tail)])


def _stripe_out(shared, bounce, out_hbm_c, base, full_chunks, tail):
    @pl.loop(0, full_chunks)
    def _(k):
        pltpu.sync_copy(shared.at[pl.ds(base + k * _CHUNK, _CHUNK)], bounce)
        pltpu.sync_copy(bounce, out_hbm_c.at[pl.ds(base + k * _CHUNK, _CHUNK)])
    if tail:
        pltpu.sync_copy(shared.at[pl.ds(base + full_chunks * _CHUNK, tail)],
                        bounce.at[pl.ds(0, tail)])
        pltpu.sync_copy(bounce.at[pl.ds(0, tail)],
                        out_hbm_c.at[pl.ds(base + full_chunks * _CHUNK, tail)])


def _make_segsum(n_pad, d, e_pad):
    epw = e_pad // _NW
    nchunks = epw // _CHUNK
    rpt = n_pad // _NSUB
    full_chunks = rpt // _CHUNK
    tail = rpt - full_chunks * _CHUNK

    scratch = [
        pltpu.VMEM((_CHUNK,), jnp.int32),
        pltpu.VMEM((_CHUNK,), jnp.int32),
        pltpu.VMEM((_CHUNK, d), jnp.float32),
        pltpu.VMEM_SHARED((n_pad, d), jnp.float32),
    ]

    def body(x_hbm, src_hbm, dst_hbm, zrow_hbm, agg_out,
             srci, dsti, rows, agg_sh):
        c = lax.axis_index("c")
        s = lax.axis_index("s")
        base = s * rpt

        pltpu.sync_copy(zrow_hbm, rows)
        _stripe_init(rows, agg_sh, base, full_chunks, tail)
        plsc.subcore_barrier()

        w = c * _NSUB + s

        @pl.loop(0, nchunks)
        def _(k):
            off = w * epw + k * _CHUNK
            pltpu.sync_copy(src_hbm.at[pl.ds(off, _CHUNK)], srci)
            pltpu.sync_copy(dst_hbm.at[pl.ds(off, _CHUNK)], dsti)
            pltpu.sync_copy(x_hbm.at[srci], rows)
            pltpu.sync_copy(rows, agg_sh.at[dsti], add=True)

        plsc.subcore_barrier()
        _stripe_out(agg_sh, rows, agg_out.at[c], base, full_chunks, tail)

    return pl.kernel(
        body,
        out_type=jax.ShapeDtypeStruct((_NCORES, n_pad, d), jnp.float32),
        mesh=_mesh(), scratch_types=scratch)


def _make_deg(n_pad, d, e_pad):
    epw = e_pad // _NW
    nchunks = epw // _CHUNK
    rpt = n_pad // _NSUB
    full_chunks = rpt // _CHUNK
    tail = rpt - full_chunks * _CHUNK

    scratch = [
        pltpu.VMEM((_CHUNK,), jnp.int32),
        pltpu.VMEM((_CHUNK, d), jnp.float32),
        pltpu.VMEM((_CHUNK, d), jnp.float32),
        pltpu.VMEM_SHARED((n_pad, d), jnp.float32),
    ]

    def body(dst_hbm, zdeg_hbm, ones_hbm, deg_out, dsti, ones, zd, deg_sh):
        c = lax.axis_index("c")
        s = lax.axis_index("s")
        base = s * rpt

        pltpu.sync_copy(zdeg_hbm, zd)
        pltpu.sync_copy(ones_hbm, ones)
        _stripe_init(zd, deg_sh, base, full_chunks, tail)
        plsc.subcore_barrier()

        w = c * _NSUB + s

        @pl.loop(0, nchunks)
        def _(k):
            off = w * epw + k * _CHUNK
            pltpu.sync_copy(dst_hbm.at[pl.ds(off, _CHUNK)], dsti)
            pltpu.sync_copy(ones, deg_sh.at[dsti], add=True)

        plsc.subcore_barrier()
        _stripe_out(deg_sh, zd, deg_out.at[c], base, full_chunks, tail)

    return pl.kernel(
        body,
        out_type=jax.ShapeDtypeStruct((_NCORES, n_pad, d), jnp.float32),
        mesh=_mesh(), scratch_types=scratch)


def _dense_layer(n, n_pad, d, partials, degp, w, bias, g, be, xx):
    def body(p_ref, deg_ref, w_ref, b_ref, g_ref, be_ref, xx_ref, o_ref):
        deg = deg_ref[0, :n, 0:1] + deg_ref[1, :n, 0:1]
        a = (p_ref[0, :n, :] + p_ref[1, :n, :]) / jnp.maximum(deg, 1.0)
        t = lax.dot_general(a, w_ref[...], (((1,), (0,)), ((), ())),
                            preferred_element_type=jnp.float32,
                            precision=lax.Precision.HIGHEST)
        t = t + b_ref[...]
        mean = jnp.mean(t, axis=0, keepdims=True)
        cen = t - mean
        var = jnp.mean(cen * cen, axis=0, keepdims=True)
        h = cen * lax.rsqrt(var + 1e-5) * g_ref[...] + be_ref[...]
        h = jnp.maximum(h, 0.0)
        nrm = jnp.sqrt(jnp.sum(h * h, axis=1, keepdims=True))
        h = h / jnp.maximum(nrm, 1e-12)
        o_ref[...] = xx_ref[...] + h

    return pl.pallas_call(
        body, out_shape=jax.ShapeDtypeStruct((n, d), jnp.float32),
    )(partials, degp, w, bias, g, be, xx)


def kernel(x, edge_index, W, b, gamma, beta):
    n, d = x.shape
    e = edge_index.shape[1]
    num_layers = W.shape[0]

    stripe = 8 * _NSUB
    n_pad = (n // stripe + 1) * stripe

    grp = _NW * _CHUNK
    e_pad = ((e + grp - 1) // grp) * grp
    pad = e_pad - e
    src = edge_index[0]
    dst = edge_index[1]
    if pad:
        src = jnp.concatenate([src, jnp.zeros((pad,), jnp.int32)])
        dst = jnp.concatenate([dst, jnp.full((pad,), n, jnp.int32)])

    zrow = jnp.zeros((_CHUNK, d), jnp.float32)
    ones = jnp.ones((_CHUNK, d), jnp.float32)

    segsum = _make_segsum(n_pad, d, e_pad)
    deg_kernel = _make_deg(n_pad, d, e_pad)

    degp = deg_kernel(dst, zrow, ones)[:, :, :8]
    xx = x
    for i in range(num_layers):
        aggp = segsum(xx, src, dst, zrow)
        xx = _dense_layer(n, n_pad, d, aggp, degp, W[i], b[i][None],
                          gamma[i][None], beta[i][None], xx)
    return xx
